# initial kernel scaffold (unmeasured)
import functools

import jax
import jax.numpy as jnp
from jax import lax
from jax.experimental import pallas as pl
from jax.experimental.pallas import tpu as pltpu

N = 16
M = 1024
D = 1024
F = 4096
FK = 4
FC = F // FK
XS = 4
CDT = jnp.bfloat16


def kernel(x, W1, W2):
    x = x.astype(CDT)
    W1 = W1.astype(CDT)
    W2 = W2.astype(CDT)

    def body(x_ref, w1_ref, w2_ref, out_ref,
             xcom, accr, accs,
             x_send_sems, x_recv_sems,
             a_send_sems, a_recv_sems,
             x_credit, a_credit):
        p = lax.axis_index("i")
        right = lax.rem(p + 1, N)
        left = lax.rem(p + N - 1, N)

        bar = pltpu.get_barrier_semaphore()
        for nbr in (left, right):
            pl.semaphore_signal(bar, inc=1, device_id=(nbr,),
                                device_id_type=pl.DeviceIdType.MESH)
        pl.semaphore_wait(bar, 2)

        def xsl(t):
            return pl.ds((t % XS) * M, M)

        def asl(s):
            return pl.ds((s % 2) * M, M)

        def contrib(xb):
            def k_step(k, c):
                h = jnp.dot(xb, w1_ref[:, pl.ds(k * FC, FC)],
                            preferred_element_type=jnp.float32)
                s = (h * jax.nn.sigmoid(h)).astype(CDT)
                return c + jnp.dot(s, w2_ref[pl.ds(k * FC, FC), :],
                                   preferred_element_type=jnp.float32)
            return lax.fori_loop(0, FK, k_step,
                                 jnp.zeros((M, D), jnp.float32))

        def x_fwd(t):
            return pltpu.make_async_remote_copy(
                src_ref=xcom.at[xsl(t), :],
                dst_ref=xcom.at[xsl(t + 1), :],
                send_sem=x_send_sems.at[t % XS],
                recv_sem=x_recv_sems.at[(t + 1) % XS],
                device_id=(right,),
                device_id_type=pl.DeviceIdType.MESH,
            )

        def x_recv(t):
            return pltpu.make_async_remote_copy(
                src_ref=xcom.at[xsl(t), :],
                dst_ref=xcom.at[xsl(t), :],
                send_sem=x_send_sems.at[t % XS],
                recv_sem=x_recv_sems.at[t % XS],
                device_id=(left,),
                device_id_type=pl.DeviceIdType.MESH,
            )

        def a_rdma(s, dev):
            return pltpu.make_async_remote_copy(
                src_ref=accs.at[asl(s), :],
                dst_ref=accr.at[asl(s), :],
                send_sem=a_send_sems.at[s % 2],
                recv_sem=a_recv_sems.at[s % 2],
                device_id=(dev,),
                device_id_type=pl.DeviceIdType.MESH,
            )

        xcom[xsl(0), :] = x_ref[:, :]
        f0 = x_fwd(0)
        f0.start()
        f0.wait_send()
        pl.semaphore_signal(x_credit, inc=1, device_id=(left,),
                            device_id_type=pl.DeviceIdType.MESH)

        acc = None
        for t in range(1, N):
            x_recv(t).wait_recv()
            fwd = None
            if t <= N - 2:
                if t >= XS - 1:
                    pl.semaphore_wait(x_credit, 1)
                fwd = x_fwd(t)
                fwd.start()
            xb = xcom[xsl(t), :]
            c = contrib(xb)
            if fwd is not None:
                fwd.wait_send()
                if t <= 11:
                    pl.semaphore_signal(x_credit, inc=1, device_id=(left,),
                                        device_id_type=pl.DeviceIdType.MESH)
            if t == 1:
                acc = c
            else:
                s_in = t - 2
                a_rdma(s_in, left).wait_recv()
                acc = accr[asl(s_in), :].astype(jnp.float32) + c
                if s_in <= 12:
                    pl.semaphore_signal(a_credit, inc=1, device_id=(left,),
                                        device_id_type=pl.DeviceIdType.MESH)
            s_out = t - 1
            if s_out >= 2:
                pl.semaphore_wait(a_credit, 1)
            accs[asl(s_out), :] = acc.astype(CDT)
            snd = a_rdma(s_out, right)
            snd.start()
            snd.wait_send()

        a_rdma(N - 2, left).wait_recv()
        c_own = contrib(x_ref[:, :])
        out_ref[:, :] = accr[asl(N - 2), :].astype(jnp.float32) + c_own

        @functools.partial(pl.run_scoped,
                           sem2=pltpu.SemaphoreType.REGULAR)
        def _(sem2):
            for nbr in (left, right):
                pl.semaphore_signal(sem2, inc=1, device_id=(nbr,),
                                    device_id_type=pl.DeviceIdType.MESH)
            pl.semaphore_wait(sem2, 2)

    return pl.pallas_call(
        body,
        out_shape=jax.ShapeDtypeStruct((M, D), jnp.float32),
        in_specs=[
            pl.BlockSpec(memory_space=pltpu.VMEM),
            pl.BlockSpec(memory_space=pltpu.VMEM),
            pl.BlockSpec(memory_space=pltpu.VMEM),
        ],
        out_specs=pl.BlockSpec(memory_space=pltpu.VMEM),
        scratch_shapes=[
            pltpu.VMEM((XS * M, D), CDT),
            pltpu.VMEM((2 * M, D), CDT),
            pltpu.VMEM((2 * M, D), CDT),
            pltpu.SemaphoreType.DMA((XS,)),
            pltpu.SemaphoreType.DMA((XS,)),
            pltpu.SemaphoreType.DMA((2,)),
            pltpu.SemaphoreType.DMA((2,)),
            pltpu.SemaphoreType.REGULAR,
            pltpu.SemaphoreType.REGULAR,
        ],
        compiler_params=pltpu.CompilerParams(collective_id=0),
    )(x, W1, W2)


# baseline (device time: 1139238 ns/iter reference)
import functools

import jax
import jax.numpy as jnp
from jax import lax
from jax.experimental import pallas as pl
from jax.experimental.pallas import tpu as pltpu

N = 16
M = 1024
D = 1024
F = 4096
FK = 16
FC = F // FK
XS = 4
CDT = jnp.bfloat16


def kernel(x, W1, W2):
    x = x.astype(CDT)
    W1 = W1.astype(CDT)
    W2 = W2.astype(CDT)

    def body(x_ref, w1_ref, w2_ref, out_ref,
             xcom, accr, accs,
             x_send_sems, x_recv_sems,
             a_send_sems, a_recv_sems,
             x_credit, a_credit):
        p = lax.axis_index("i")
        right = lax.rem(p + 1, N)
        left = lax.rem(p + N - 1, N)

        bar = pltpu.get_barrier_semaphore()
        for nbr in (left, right):
            pl.semaphore_signal(bar, inc=1, device_id=(nbr,),
                                device_id_type=pl.DeviceIdType.MESH)
        pl.semaphore_wait(bar, 2)

        def xsl(t):
            return pl.ds((t % XS) * M, M)

        def asl(s):
            return pl.ds((s % 2) * M, M)

        def contrib(xb, init):
            def k_step(k, c):
                h = jnp.dot(xb, w1_ref[:, pl.ds(k * FC, FC)],
                            preferred_element_type=jnp.float32)
                s = (h * jax.nn.sigmoid(h)).astype(CDT)
                return c + jnp.dot(s, w2_ref[pl.ds(k * FC, FC), :],
                                   preferred_element_type=jnp.float32)
            return lax.fori_loop(0, FK, k_step, init)

        def x_fwd(t):
            return pltpu.make_async_remote_copy(
                src_ref=xcom.at[xsl(t), :],
                dst_ref=xcom.at[xsl(t + 1), :],
                send_sem=x_send_sems.at[t % XS],
                recv_sem=x_recv_sems.at[(t + 1) % XS],
                device_id=(right,),
                device_id_type=pl.DeviceIdType.MESH,
            )

        def x_recv(t):
            return pltpu.make_async_remote_copy(
                src_ref=xcom.at[xsl(t), :],
                dst_ref=xcom.at[xsl(t), :],
                send_sem=x_send_sems.at[t % XS],
                recv_sem=x_recv_sems.at[t % XS],
                device_id=(left,),
                device_id_type=pl.DeviceIdType.MESH,
            )

        def a_rdma(s, dev):
            return pltpu.make_async_remote_copy(
                src_ref=accs.at[asl(s), :],
                dst_ref=accr.at[asl(s), :],
                send_sem=a_send_sems.at[s % 2],
                recv_sem=a_recv_sems.at[s % 2],
                device_id=(dev,),
                device_id_type=pl.DeviceIdType.MESH,
            )

        xcom[xsl(0), :] = x_ref[:, :]
        f0 = x_fwd(0)
        f0.start()
        f0.wait_send()
        pl.semaphore_signal(x_credit, inc=1, device_id=(left,),
                            device_id_type=pl.DeviceIdType.MESH)

        acc = None
        for t in range(1, N):
            x_recv(t).wait_recv()
            fwd = None
            if t <= N - 2:
                if t >= XS - 1:
                    pl.semaphore_wait(x_credit, 1)
                fwd = x_fwd(t)
                fwd.start()
            if t == 1:
                init = jnp.zeros((M, D), jnp.float32)
            else:
                s_in = t - 2
                a_rdma(s_in, left).wait_recv()
                if s_in <= 12:
                    pl.semaphore_signal(a_credit, inc=1, device_id=(left,),
                                        device_id_type=pl.DeviceIdType.MESH)
                init = accr[asl(s_in), :].astype(jnp.float32)
            xb = xcom[xsl(t), :]
            acc = contrib(xb, init)
            if fwd is not None:
                fwd.wait_send()
                if t <= 11:
                    pl.semaphore_signal(x_credit, inc=1, device_id=(left,),
                                        device_id_type=pl.DeviceIdType.MESH)
            s_out = t - 1
            if s_out >= 2:
                pl.semaphore_wait(a_credit, 1)
            accs[asl(s_out), :] = acc.astype(CDT)
            snd = a_rdma(s_out, right)
            snd.start()
            snd.wait_send()

        a_rdma(N - 2, left).wait_recv()
        out_ref[:, :] = contrib(x_ref[:, :],
                                accr[asl(N - 2), :].astype(jnp.float32))

        @functools.partial(pl.run_scoped,
                           sem2=pltpu.SemaphoreType.REGULAR)
        def _(sem2):
            for nbr in (left, right):
                pl.semaphore_signal(sem2, inc=1, device_id=(nbr,),
                                    device_id_type=pl.DeviceIdType.MESH)
            pl.semaphore_wait(sem2, 2)

    return pl.pallas_call(
        body,
        out_shape=jax.ShapeDtypeStruct((M, D), jnp.float32),
        in_specs=[
            pl.BlockSpec(memory_space=pltpu.VMEM),
            pl.BlockSpec(memory_space=pltpu.VMEM),
            pl.BlockSpec(memory_space=pltpu.VMEM),
        ],
        out_specs=pl.BlockSpec(memory_space=pltpu.VMEM),
        scratch_shapes=[
            pltpu.VMEM((XS * M, D), CDT),
            pltpu.VMEM((2 * M, D), CDT),
            pltpu.VMEM((2 * M, D), CDT),
            pltpu.SemaphoreType.DMA((XS,)),
            pltpu.SemaphoreType.DMA((XS,)),
            pltpu.SemaphoreType.DMA((2,)),
            pltpu.SemaphoreType.DMA((2,)),
            pltpu.SemaphoreType.REGULAR,
            pltpu.SemaphoreType.REGULAR,
        ],
        compiler_params=pltpu.CompilerParams(
            collective_id=0,
            vmem_limit_bytes=30 * 1024 * 1024,
        ),
    )(x, W1, W2)


# device time: 856228 ns/iter; 1.3305x vs baseline; 1.3305x over previous
import functools

import jax
import jax.numpy as jnp
from jax import lax
from jax.experimental import pallas as pl
from jax.experimental.pallas import tpu as pltpu

N = 16
M = 1024
M2 = 512
D = 1024
F = 4096
FK = 16
FC = F // FK
XS = 3
CDT = jnp.bfloat16


def kernel(x, W1, W2):
    x = x.astype(CDT)
    W1 = W1.astype(CDT)
    W2 = W2.astype(CDT)

    def body(x_ref, w1_ref, w2_ref, out_ref, *sc):
        (xcom0, accr0, accs0, cbuf0,
         xcom1, accr1, accs1, cbuf1,
         xss0, xrs0, ass0, ars0,
         xss1, xrs1, ass1, ars1,
         xcred0, acred0, xcred1, acred1) = sc
        p = lax.axis_index("i")
        right = lax.rem(p + 1, N)
        left = lax.rem(p + N - 1, N)

        bar = pltpu.get_barrier_semaphore()
        for nbr in (left, right):
            pl.semaphore_signal(bar, inc=1, device_id=(nbr,),
                                device_id_type=pl.DeviceIdType.MESH)
        pl.semaphore_wait(bar, 2)

        dirs = (
            (xcom0, accr0, accs0, cbuf0, xss0, xrs0, ass0, ars0,
             xcred0, acred0, right, left, 0),
            (xcom1, accr1, accs1, cbuf1, xss1, xrs1, ass1, ars1,
             xcred1, acred1, left, right, M2),
        )

        def xsl(t):
            return pl.ds((t % XS) * M2, M2)

        def asl(s):
            return pl.ds((s % 2) * M2, M2)

        def contrib(xb):
            def k_step(k, c):
                h = jnp.dot(xb, w1_ref[:, pl.ds(k * FC, FC)],
                            preferred_element_type=jnp.float32)
                s = (h * jax.nn.sigmoid(h)).astype(CDT)
                return c + jnp.dot(s, w2_ref[pl.ds(k * FC, FC), :],
                                   preferred_element_type=jnp.float32)
            return lax.fori_loop(0, FK, k_step,
                                 jnp.zeros((M2, D), jnp.float32))

        def x_fwd(dc, h):
            return pltpu.make_async_remote_copy(
                src_ref=dc[0].at[xsl(h), :],
                dst_ref=dc[0].at[xsl(h + 1), :],
                send_sem=dc[4].at[h % XS],
                recv_sem=dc[5].at[(h + 1) % XS],
                device_id=(dc[10],),
                device_id_type=pl.DeviceIdType.MESH,
            )

        def x_rcv(dc, h):
            return pltpu.make_async_remote_copy(
                src_ref=dc[0].at[xsl(h), :],
                dst_ref=dc[0].at[xsl(h), :],
                send_sem=dc[4].at[h % XS],
                recv_sem=dc[5].at[h % XS],
                device_id=(dc[11],),
                device_id_type=pl.DeviceIdType.MESH,
            )

        def a_rdma(dc, s, dev):
            return pltpu.make_async_remote_copy(
                src_ref=dc[2].at[asl(s), :],
                dst_ref=dc[1].at[asl(s), :],
                send_sem=dc[6].at[s % 2],
                recv_sem=dc[7].at[s % 2],
                device_id=(dev,),
                device_id_type=pl.DeviceIdType.MESH,
            )

        def sig(sem, dev):
            pl.semaphore_signal(sem, inc=1, device_id=(dev,),
                                device_id_type=pl.DeviceIdType.MESH)

        fwd_desc = [[None] * (N - 1) for _ in range(2)]
        asend_desc = [[None] * (N - 1) for _ in range(2)]

        for dc in dirs:
            dc[0][xsl(0), :] = x_ref[pl.ds(dc[12], M2), :]
        for d, dc in enumerate(dirs):
            f = x_fwd(dc, 0)
            f.start()
            fwd_desc[d][0] = f
        for dc in dirs:
            x_rcv(dc, 1).wait_recv()
        for d, dc in enumerate(dirs):
            f = x_fwd(dc, 1)
            f.start()
            fwd_desc[d][1] = f
        for dc in dirs:
            dc[3][:, :] = contrib(dc[0][xsl(1), :]).astype(CDT)
        for d, dc in enumerate(dirs):
            fwd_desc[d][0].wait_send()
            sig(dc[8], dc[11])

        for t in range(1, N):
            for d, dc in enumerate(dirs):
                if t >= 2:
                    s_in = t - 2
                    a_rdma(dc, s_in, dc[11]).wait_recv()
                    if s_in <= 12:
                        sig(dc[9], dc[11])
                    accv = (dc[3][:, :].astype(jnp.float32)
                            + dc[1][asl(s_in), :].astype(jnp.float32))
                else:
                    accv = dc[3][:, :].astype(jnp.float32)
                s_out = t - 1
                if s_out >= 2:
                    asend_desc[d][s_out - 2].wait_send()
                    pl.semaphore_wait(dc[9], 1)
                dc[2][asl(s_out), :] = accv.astype(CDT)
                snd = a_rdma(dc, s_out, dc[10])
                snd.start()
                asend_desc[d][s_out] = snd
            if t <= N - 2:
                for d, dc in enumerate(dirs):
                    x_rcv(dc, t + 1).wait_recv()
                    if t + 1 <= N - 2:
                        if t + 1 >= XS - 1:
                            pl.semaphore_wait(dc[8], 1)
                        f = x_fwd(dc, t + 1)
                        f.start()
                        fwd_desc[d][t + 1] = f
                for d, dc in enumerate(dirs):
                    fwd_desc[d][t].wait_send()
                    if t <= N - 2 - (XS - 1):
                        sig(dc[8], dc[11])
                    dc[3][:, :] = contrib(dc[0][xsl(t + 1), :]).astype(CDT)
            else:
                for dc in dirs:
                    dc[3][:, :] = contrib(
                        x_ref[pl.ds(dc[12], M2), :]).astype(CDT)

        for d, dc in enumerate(dirs):
            a_rdma(dc, N - 2, dc[11]).wait_recv()
            out_ref[pl.ds(dc[12], M2), :] = (
                dc[3][:, :].astype(jnp.float32)
                + dc[1][asl(N - 2), :].astype(jnp.float32))
            asend_desc[d][N - 3].wait_send()
            asend_desc[d][N - 2].wait_send()

        @functools.partial(pl.run_scoped,
                           sem2=pltpu.SemaphoreType.REGULAR)
        def _(sem2):
            for nbr in (left, right):
                pl.semaphore_signal(sem2, inc=1, device_id=(nbr,),
                                    device_id_type=pl.DeviceIdType.MESH)
            pl.semaphore_wait(sem2, 2)

    return pl.pallas_call(
        body,
        out_shape=jax.ShapeDtypeStruct((M, D), jnp.float32),
        in_specs=[
            pl.BlockSpec(memory_space=pltpu.VMEM),
            pl.BlockSpec(memory_space=pltpu.VMEM),
            pl.BlockSpec(memory_space=pltpu.VMEM),
        ],
        out_specs=pl.BlockSpec(memory_space=pltpu.VMEM),
        scratch_shapes=[
            pltpu.VMEM((XS * M2, D), CDT),
            pltpu.VMEM((2 * M2, D), CDT),
            pltpu.VMEM((2 * M2, D), CDT),
            pltpu.VMEM((M2, D), CDT),
            pltpu.VMEM((XS * M2, D), CDT),
            pltpu.VMEM((2 * M2, D), CDT),
            pltpu.VMEM((2 * M2, D), CDT),
            pltpu.VMEM((M2, D), CDT),
            pltpu.SemaphoreType.DMA((XS,)),
            pltpu.SemaphoreType.DMA((XS,)),
            pltpu.SemaphoreType.DMA((2,)),
            pltpu.SemaphoreType.DMA((2,)),
            pltpu.SemaphoreType.DMA((XS,)),
            pltpu.SemaphoreType.DMA((XS,)),
            pltpu.SemaphoreType.DMA((2,)),
            pltpu.SemaphoreType.DMA((2,)),
            pltpu.SemaphoreType.REGULAR,
            pltpu.SemaphoreType.REGULAR,
            pltpu.SemaphoreType.REGULAR,
            pltpu.SemaphoreType.REGULAR,
        ],
        compiler_params=pltpu.CompilerParams(
            collective_id=0,
            vmem_limit_bytes=24 * 1024 * 1024,
        ),
    )(x, W1, W2)


# device time: 778949 ns/iter; 1.4625x vs baseline; 1.0992x over previous
import functools

import jax
import jax.numpy as jnp
from jax import lax
from jax.experimental import pallas as pl
from jax.experimental.pallas import tpu as pltpu

N = 16
M = 1024
M2 = 512
D = 1024
F = 4096
FK = 16
FC = F // FK
XS = 3
CDT = jnp.bfloat16


def kernel(x, W1, W2):
    x = x.astype(CDT)
    W1 = W1.astype(CDT)
    W2 = W2.astype(CDT)

    def body(x_ref, w1_ref, w2_ref, out_ref, *sc):
        (xcom0, accr0, accs0,
         xcom1, accr1, accs1, cbuf,
         xss0, xrs0, ass0, ars0,
         xss1, xrs1, ass1, ars1,
         xcred0, acred0, xcred1, acred1) = sc
        p = lax.axis_index("i")
        right = lax.rem(p + 1, N)
        left = lax.rem(p + N - 1, N)

        bar = pltpu.get_barrier_semaphore()
        for nbr in (left, right):
            pl.semaphore_signal(bar, inc=1, device_id=(nbr,),
                                device_id_type=pl.DeviceIdType.MESH)
        pl.semaphore_wait(bar, 2)

        dirs = (
            (xcom0, accr0, accs0, xss0, xrs0, ass0, ars0,
             xcred0, acred0, right, left, 0),
            (xcom1, accr1, accs1, xss1, xrs1, ass1, ars1,
             xcred1, acred1, left, right, M2),
        )

        def xsl(t):
            return pl.ds((t % XS) * M2, M2)

        def asl(s):
            return pl.ds((s % 2) * M2, M2)

        def contrib(xb):
            def k_step(k, c):
                h = jnp.dot(xb, w1_ref[:, pl.ds(k * FC, FC)],
                            preferred_element_type=jnp.float32)
                s = (h * jax.nn.sigmoid(h)).astype(CDT)
                return c + jnp.dot(s, w2_ref[pl.ds(k * FC, FC), :],
                                   preferred_element_type=jnp.float32)
            return lax.fori_loop(0, FK, k_step,
                                 jnp.zeros((M, D), jnp.float32))

        def compute_into_cbuf(t):
            xcat = jnp.concatenate(
                [xcom0[xsl(t), :], xcom1[xsl(t), :]], axis=0)
            cbuf[:, :] = contrib(xcat).astype(CDT)

        def x_fwd(dc, h):
            return pltpu.make_async_remote_copy(
                src_ref=dc[0].at[xsl(h), :],
                dst_ref=dc[0].at[xsl(h + 1), :],
                send_sem=dc[3].at[h % XS],
                recv_sem=dc[4].at[(h + 1) % XS],
                device_id=(dc[9],),
                device_id_type=pl.DeviceIdType.MESH,
            )

        def x_rcv(dc, h):
            return pltpu.make_async_remote_copy(
                src_ref=dc[0].at[xsl(h), :],
                dst_ref=dc[0].at[xsl(h), :],
                send_sem=dc[3].at[h % XS],
                recv_sem=dc[4].at[h % XS],
                device_id=(dc[10],),
                device_id_type=pl.DeviceIdType.MESH,
            )

        def a_rdma(dc, s, dev):
            return pltpu.make_async_remote_copy(
                src_ref=dc[2].at[asl(s), :],
                dst_ref=dc[1].at[asl(s), :],
                send_sem=dc[5].at[s % 2],
                recv_sem=dc[6].at[s % 2],
                device_id=(dev,),
                device_id_type=pl.DeviceIdType.MESH,
            )

        def sig(sem, dev):
            pl.semaphore_signal(sem, inc=1, device_id=(dev,),
                                device_id_type=pl.DeviceIdType.MESH)

        fwd_desc = [[None] * (N - 1) for _ in range(2)]
        asend_desc = [[None] * (N - 1) for _ in range(2)]

        for dc in dirs:
            dc[0][xsl(0), :] = x_ref[pl.ds(dc[11], M2), :]
        for d, dc in enumerate(dirs):
            f = x_fwd(dc, 0)
            f.start()
            fwd_desc[d][0] = f
        for dc in dirs:
            x_rcv(dc, 1).wait_recv()
        for d, dc in enumerate(dirs):
            f = x_fwd(dc, 1)
            f.start()
            fwd_desc[d][1] = f
        compute_into_cbuf(1)
        for d, dc in enumerate(dirs):
            fwd_desc[d][0].wait_send()
            sig(dc[7], dc[10])

        for t in range(1, N):
            for d, dc in enumerate(dirs):
                rows = pl.ds(dc[11], M2)
                if t >= 2:
                    s_in = t - 2
                    a_rdma(dc, s_in, dc[10]).wait_recv()
                    if s_in <= 12:
                        sig(dc[8], dc[10])
                    accv = (cbuf[rows, :].astype(jnp.float32)
                            + dc[1][asl(s_in), :].astype(jnp.float32))
                else:
                    accv = cbuf[rows, :].astype(jnp.float32)
                s_out = t - 1
                if s_out >= 2:
                    asend_desc[d][s_out - 2].wait_send()
                    pl.semaphore_wait(dc[8], 1)
                dc[2][asl(s_out), :] = accv.astype(CDT)
                snd = a_rdma(dc, s_out, dc[9])
                snd.start()
                asend_desc[d][s_out] = snd
            if t <= N - 2:
                for d, dc in enumerate(dirs):
                    x_rcv(dc, t + 1).wait_recv()
                    if t + 1 <= N - 2:
                        if t + 1 >= XS - 1:
                            pl.semaphore_wait(dc[7], 1)
                        f = x_fwd(dc, t + 1)
                        f.start()
                        fwd_desc[d][t + 1] = f
                for d, dc in enumerate(dirs):
                    fwd_desc[d][t].wait_send()
                    if t <= N - 2 - (XS - 1):
                        sig(dc[7], dc[10])
                compute_into_cbuf(t + 1)
            else:
                cbuf[:, :] = contrib(x_ref[:, :]).astype(CDT)

        for d, dc in enumerate(dirs):
            a_rdma(dc, N - 2, dc[10]).wait_recv()
            out_ref[pl.ds(dc[11], M2), :] = (
                cbuf[pl.ds(dc[11], M2), :].astype(jnp.float32)
                + dc[1][asl(N - 2), :].astype(jnp.float32))
            asend_desc[d][N - 3].wait_send()
            asend_desc[d][N - 2].wait_send()

        @functools.partial(pl.run_scoped,
                           sem2=pltpu.SemaphoreType.REGULAR)
        def _(sem2):
            for nbr in (left, right):
                pl.semaphore_signal(sem2, inc=1, device_id=(nbr,),
                                    device_id_type=pl.DeviceIdType.MESH)
            pl.semaphore_wait(sem2, 2)

    return pl.pallas_call(
        body,
        out_shape=jax.ShapeDtypeStruct((M, D), jnp.float32),
        in_specs=[
            pl.BlockSpec(memory_space=pltpu.VMEM),
            pl.BlockSpec(memory_space=pltpu.VMEM),
            pl.BlockSpec(memory_space=pltpu.VMEM),
        ],
        out_specs=pl.BlockSpec(memory_space=pltpu.VMEM),
        scratch_shapes=[
            pltpu.VMEM((XS * M2, D), CDT),
            pltpu.VMEM((2 * M2, D), CDT),
            pltpu.VMEM((2 * M2, D), CDT),
            pltpu.VMEM((XS * M2, D), CDT),
            pltpu.VMEM((2 * M2, D), CDT),
            pltpu.VMEM((2 * M2, D), CDT),
            pltpu.VMEM((M, D), CDT),
            pltpu.SemaphoreType.DMA((XS,)),
            pltpu.SemaphoreType.DMA((XS,)),
            pltpu.SemaphoreType.DMA((2,)),
            pltpu.SemaphoreType.DMA((2,)),
            pltpu.SemaphoreType.DMA((XS,)),
            pltpu.SemaphoreType.DMA((XS,)),
            pltpu.SemaphoreType.DMA((2,)),
            pltpu.SemaphoreType.DMA((2,)),
            pltpu.SemaphoreType.REGULAR,
            pltpu.SemaphoreType.REGULAR,
            pltpu.SemaphoreType.REGULAR,
            pltpu.SemaphoreType.REGULAR,
        ],
        compiler_params=pltpu.CompilerParams(
            collective_id=0,
            vmem_limit_bytes=30 * 1024 * 1024,
        ),
    )(x, W1, W2)
